# baseline (speedup 1.0000x reference)
import functools

import jax
import jax.numpy as jnp
from jax import lax
from jax.experimental import pallas as pl
from jax.experimental.pallas import tpu as pltpu
from jax.experimental.pallas import tpu_sc as plsc

_NC, _NS, _LANES = 2, 16, 16
_NW = _NC * _NS

_L_SC = 12288
_TL = 1024


def _make_sc_call(B, A, N, LSC):
    G = B * A
    CL = LSC // _NW
    assert LSC % _NW == 0 and CL % 128 == 0

    mesh = plsc.VectorSubcoreMesh(
        core_axis_name="c", subcore_axis_name="s",
        num_cores=_NC, num_subcores=_NS)

    @functools.partial(
        pl.kernel,
        out_type=(
            jax.ShapeDtypeStruct((B, A, LSC), jnp.float32),
            jax.ShapeDtypeStruct((B, A, LSC), jnp.int32),
        ),
        mesh=mesh,
        scratch_types=[
            pltpu.VMEM((B, CL), jnp.float32),
            pltpu.VMEM((2, N, CL), jnp.float32),
            pltpu.VMEM((B, A, CL), jnp.float32),
            pltpu.VMEM((B, A, CL), jnp.int32),
            pltpu.SemaphoreType.DMA,
            pltpu.SemaphoreType.DMA,
        ],
    )
    def sc_call(mixed_hbm, ref_hbm, omax_hbm, oidx_hbm, m_v, r_v, omax_v,
                oidx_v, sr0, sr1):
        srs = (sr0, sr1)
        wid = lax.axis_index("s") * _NC + lax.axis_index("c")
        col0 = wid * CL
        pltpu.sync_copy(mixed_hbm.at[:, pl.ds(col0, CL)], m_v)

        def ref_copy(g, k):
            return pltpu.make_async_copy(
                ref_hbm.at[pl.ds(g * N, N), pl.ds(col0, CL)],
                r_v.at[k], srs[k])

        ref_copy(0, 0).start()

        @pl.loop(0, G // 2)
        def _(t):
            for k in range(2):
                g = 2 * t + k
                gn = jnp.minimum(g + 1, G - 1)
                ref_copy(gn, 1 - k).start()
                ref_copy(g, k).wait()
                b = g // A
                a = g - b * A

                @pl.loop(0, CL // _LANES, unroll=8)
                def _(j):
                    sl = pl.ds(j * _LANES, _LANES)
                    mv = m_v[b, sl]
                    best = mv * r_v[k, 0, sl]
                    besti = jnp.zeros((_LANES,), jnp.int32)
                    for n in range(1, N):
                        p = mv * r_v[k, n, sl]
                        gt = p > best
                        besti = jnp.where(gt, jnp.int32(n), besti)
                        best = jnp.maximum(best, p)
                    omax_v[b, a, sl] = best
                    oidx_v[b, a, sl] = besti

        ref_copy(G - 1, 0).wait()
        h1 = pltpu.async_copy(omax_v, omax_hbm.at[:, :, pl.ds(col0, CL)], sr0)
        h2 = pltpu.async_copy(oidx_v, oidx_hbm.at[:, :, pl.ds(col0, CL)], sr1)
        h1.wait()
        h2.wait()

    return sc_call


def _tc_body(m_ref, r_ref, omax_ref, oidx_ref, *, N):
    r = r_ref[0]
    p = m_ref[0] * r
    mx = jnp.max(p, axis=0, keepdims=True)
    row = lax.broadcasted_iota(jnp.int32, p.shape, 0)
    idx = jnp.min(jnp.where(p == mx, row, N), axis=0, keepdims=True)
    omax_ref[0] = mx
    oidx_ref[0] = idx


def _make_tc_call(B, A, N, L, LSC, G):
    LTC = L - LSC
    assert LTC % _TL == 0
    off = LSC // _TL
    grid = (G, LTC // _TL)

    return pl.pallas_call(
        functools.partial(_tc_body, N=N),
        grid=grid,
        in_specs=[
            pl.BlockSpec((1, 1, _TL), lambda g, c: (g // A, 0, c + off)),
            pl.BlockSpec((1, N, _TL), lambda g, c: (g, 0, c + off)),
        ],
        out_specs=[
            pl.BlockSpec((1, 1, _TL), lambda g, c: (g, 0, c)),
            pl.BlockSpec((1, 1, _TL), lambda g, c: (g, 0, c)),
        ],
        out_shape=(
            jax.ShapeDtypeStruct((G, 1, LTC), jnp.float32),
            jax.ShapeDtypeStruct((G, 1, LTC), jnp.int32),
        ),
    )


def kernel(input_mixed, ref_panel):
    B, A, N, L = ref_panel.shape
    G = B * A
    sc_call = _make_sc_call(B, A, N, _L_SC)
    tc_call = _make_tc_call(B, A, N, L, _L_SC, G)
    ref2d = ref_panel.reshape(G * N, L)
    sc_max, sc_idx = sc_call(input_mixed, ref2d)
    tc_max, tc_idx = tc_call(input_mixed.reshape(B, 1, L),
                             ref_panel.reshape(G, N, L))
    omax = jnp.concatenate([sc_max, tc_max.reshape(B, A, L - _L_SC)], axis=2)
    oidx = jnp.concatenate([sc_idx, tc_idx.reshape(B, A, L - _L_SC)], axis=2)
    return omax, oidx


# speedup vs baseline: 2.1322x; 2.1322x over previous
import functools

import jax
import jax.numpy as jnp
from jax import lax
from jax.experimental import pallas as pl
from jax.experimental.pallas import tpu as pltpu
from jax.experimental.pallas import tpu_sc as plsc

_NC, _NS, _LANES = 2, 16, 16
_NW = _NC * _NS

_L_SC = 12288
_TL = 4096


def _make_sc_call(B, A, N, LSC):
    G = B * A
    CL = LSC // _NW
    assert LSC % _NW == 0 and CL % 128 == 0

    mesh = plsc.VectorSubcoreMesh(
        core_axis_name="c", subcore_axis_name="s",
        num_cores=_NC, num_subcores=_NS)

    @functools.partial(
        pl.kernel,
        out_type=(
            jax.ShapeDtypeStruct((B, A, LSC), jnp.float32),
            jax.ShapeDtypeStruct((B, A, LSC), jnp.int32),
        ),
        mesh=mesh,
        scratch_types=[
            pltpu.VMEM((B, CL), jnp.float32),
            pltpu.VMEM((2, N, CL), jnp.float32),
            pltpu.VMEM((B, A, CL), jnp.float32),
            pltpu.VMEM((B, A, CL), jnp.int32),
            pltpu.SemaphoreType.DMA,
            pltpu.SemaphoreType.DMA,
        ],
    )
    def sc_call(mixed_hbm, ref_hbm, omax_hbm, oidx_hbm, m_v, r_v, omax_v,
                oidx_v, sr0, sr1):
        srs = (sr0, sr1)
        wid = lax.axis_index("s") * _NC + lax.axis_index("c")
        col0 = wid * CL
        pltpu.sync_copy(mixed_hbm.at[:, pl.ds(col0, CL)], m_v)

        def ref_copy(g, k):
            return pltpu.make_async_copy(
                ref_hbm.at[pl.ds(g * N, N), pl.ds(col0, CL)],
                r_v.at[k], srs[k])

        ref_copy(0, 0).start()

        @pl.loop(0, G // 2)
        def _(t):
            for k in range(2):
                g = 2 * t + k
                gn = jnp.minimum(g + 1, G - 1)
                ref_copy(gn, 1 - k).start()
                ref_copy(g, k).wait()
                b = g // A
                a = g - b * A

                @pl.loop(0, CL // _LANES, unroll=8)
                def _(j):
                    sl = pl.ds(j * _LANES, _LANES)
                    mv = m_v[b, sl]
                    best = mv * r_v[k, 0, sl]
                    besti = jnp.zeros((_LANES,), jnp.int32)
                    for n in range(1, N):
                        p = mv * r_v[k, n, sl]
                        gt = p > best
                        besti = jnp.where(gt, jnp.int32(n), besti)
                        best = jnp.maximum(best, p)
                    omax_v[b, a, sl] = best
                    oidx_v[b, a, sl] = besti

        ref_copy(G - 1, 0).wait()
        h1 = pltpu.async_copy(omax_v, omax_hbm.at[:, :, pl.ds(col0, CL)], sr0)
        h2 = pltpu.async_copy(oidx_v, oidx_hbm.at[:, :, pl.ds(col0, CL)], sr1)
        h1.wait()
        h2.wait()

    return sc_call


def _tc_body(m_ref, r_ref, omax_ref, oidx_ref, *, N):
    r = r_ref[0]
    p = m_ref[0] * r
    mx = jnp.max(p, axis=0, keepdims=True)
    row = lax.broadcasted_iota(jnp.int32, p.shape, 0)
    idx = jnp.min(jnp.where(p == mx, row, N), axis=0, keepdims=True)
    omax_ref[0] = mx
    oidx_ref[0] = idx


def _make_tc_call(B, A, N, L, LSC, G):
    LTC = L - LSC
    assert LTC % _TL == 0
    off = LSC // _TL
    grid = (G, LTC // _TL)

    return pl.pallas_call(
        functools.partial(_tc_body, N=N),
        grid=grid,
        in_specs=[
            pl.BlockSpec((1, 1, _TL), lambda g, c: (g // A, 0, c + off)),
            pl.BlockSpec((1, N, _TL), lambda g, c: (g, 0, c + off)),
        ],
        out_specs=[
            pl.BlockSpec((1, 1, _TL), lambda g, c: (g, 0, c)),
            pl.BlockSpec((1, 1, _TL), lambda g, c: (g, 0, c)),
        ],
        out_shape=(
            jax.ShapeDtypeStruct((G, 1, LTC), jnp.float32),
            jax.ShapeDtypeStruct((G, 1, LTC), jnp.int32),
        ),
    )


def kernel(input_mixed, ref_panel):
    B, A, N, L = ref_panel.shape
    G = B * A
    sc_call = _make_sc_call(B, A, N, _L_SC)
    tc_call = _make_tc_call(B, A, N, L, _L_SC, G)
    ref2d = ref_panel.reshape(G * N, L)
    sc_max, sc_idx = sc_call(input_mixed, ref2d)
    tc_max, tc_idx = tc_call(input_mixed.reshape(B, 1, L),
                             ref_panel.reshape(G, N, L))
    omax = jnp.concatenate([sc_max, tc_max.reshape(B, A, L - _L_SC)], axis=2)
    oidx = jnp.concatenate([sc_idx, tc_idx.reshape(B, A, L - _L_SC)], axis=2)
    return omax, oidx


# speedup vs baseline: 2.1665x; 1.0161x over previous
import functools

import jax
import jax.numpy as jnp
from jax import lax
from jax.experimental import pallas as pl
from jax.experimental.pallas import tpu as pltpu
from jax.experimental.pallas import tpu_sc as plsc

_NC, _NS, _LANES = 2, 16, 16
_NW = _NC * _NS

_L_SC = 16384
_TL = 4096


def _make_sc_call(B, A, N, LSC):
    G = B * A
    CL = LSC // _NW
    assert LSC % _NW == 0 and CL % 128 == 0

    mesh = plsc.VectorSubcoreMesh(
        core_axis_name="c", subcore_axis_name="s",
        num_cores=_NC, num_subcores=_NS)

    @functools.partial(
        pl.kernel,
        out_type=(
            jax.ShapeDtypeStruct((B, A, LSC), jnp.float32),
            jax.ShapeDtypeStruct((B, A, LSC), jnp.int32),
        ),
        mesh=mesh,
        scratch_types=[
            pltpu.VMEM((B, CL), jnp.float32),
            pltpu.VMEM((2, N, CL), jnp.float32),
            pltpu.VMEM((B, A, CL), jnp.float32),
            pltpu.VMEM((B, A, CL), jnp.int32),
            pltpu.SemaphoreType.DMA,
            pltpu.SemaphoreType.DMA,
        ],
    )
    def sc_call(mixed_hbm, ref_hbm, omax_hbm, oidx_hbm, m_v, r_v, omax_v,
                oidx_v, sr0, sr1):
        srs = (sr0, sr1)
        wid = lax.axis_index("s") * _NC + lax.axis_index("c")
        col0 = wid * CL
        pltpu.sync_copy(mixed_hbm.at[:, pl.ds(col0, CL)], m_v)

        def ref_copy(g, k):
            return pltpu.make_async_copy(
                ref_hbm.at[pl.ds(g * N, N), pl.ds(col0, CL)],
                r_v.at[k], srs[k])

        ref_copy(0, 0).start()

        @pl.loop(0, G // 2)
        def _(t):
            for k in range(2):
                g = 2 * t + k
                gn = jnp.minimum(g + 1, G - 1)
                ref_copy(gn, 1 - k).start()
                ref_copy(g, k).wait()
                b = g // A
                a = g - b * A

                @pl.loop(0, CL // _LANES, unroll=4)
                def _(j):
                    sl = pl.ds(j * _LANES, _LANES)
                    mv = m_v[b, sl]
                    best = mv * r_v[k, 0, sl]
                    besti = jnp.zeros((_LANES,), jnp.int32)
                    for n in range(1, N):
                        p = mv * r_v[k, n, sl]
                        gt = p > best
                        besti = jnp.where(gt, jnp.int32(n), besti)
                        best = jnp.maximum(best, p)
                    omax_v[b, a, sl] = best
                    oidx_v[b, a, sl] = besti

        ref_copy(G - 1, 0).wait()
        h1 = pltpu.async_copy(omax_v, omax_hbm.at[:, :, pl.ds(col0, CL)], sr0)
        h2 = pltpu.async_copy(oidx_v, oidx_hbm.at[:, :, pl.ds(col0, CL)], sr1)
        h1.wait()
        h2.wait()

    return sc_call


def _tc_body(m_ref, r_ref, omax_ref, oidx_ref, *, N, A):
    g = pl.program_id(0)
    b = g // A
    m4 = m_ref[...]
    mrow = jnp.where(
        b == 0, m4[0:1],
        jnp.where(b == 1, m4[1:2], jnp.where(b == 2, m4[2:3], m4[3:4])))
    r = r_ref[0]
    p = mrow * r
    mx = jnp.max(p, axis=0, keepdims=True)
    row = lax.broadcasted_iota(jnp.int32, p.shape, 0)
    idx = jnp.min(jnp.where(p == mx, row, N), axis=0, keepdims=True)
    omax_ref[0] = mx
    oidx_ref[0] = idx


def _make_tc_call(B, A, N, L, LSC, G):
    LTC = L - LSC
    assert LTC % _TL == 0
    off = LSC // _TL
    grid = (G, LTC // _TL)

    return pl.pallas_call(
        functools.partial(_tc_body, N=N, A=A),
        grid=grid,
        in_specs=[
            pl.BlockSpec((B, _TL), lambda g, c: (0, c + off)),
            pl.BlockSpec((1, N, _TL), lambda g, c: (g, 0, c + off)),
        ],
        out_specs=[
            pl.BlockSpec((1, 1, _TL), lambda g, c: (g, 0, c + off)),
            pl.BlockSpec((1, 1, _TL), lambda g, c: (g, 0, c + off)),
        ],
        out_shape=(
            jax.ShapeDtypeStruct((G, 1, L), jnp.float32),
            jax.ShapeDtypeStruct((G, 1, L), jnp.int32),
        ),
    )


def kernel(input_mixed, ref_panel):
    B, A, N, L = ref_panel.shape
    G = B * A
    sc_call = _make_sc_call(B, A, N, _L_SC)
    tc_call = _make_tc_call(B, A, N, L, _L_SC, G)
    ref2d = ref_panel.reshape(G * N, L)
    sc_max, sc_idx = sc_call(input_mixed, ref2d)
    tc_max, tc_idx = tc_call(input_mixed, ref_panel.reshape(G, N, L))
    omax = lax.dynamic_update_slice(
        tc_max.reshape(B, A, L), sc_max, (0, 0, 0))
    oidx = lax.dynamic_update_slice(
        tc_idx.reshape(B, A, L), sc_idx, (0, 0, 0))
    return omax, oidx


# speedup vs baseline: 2.9910x; 1.3806x over previous
import functools

import jax
import jax.numpy as jnp
from jax import lax
from jax.experimental import pallas as pl
from jax.experimental.pallas import tpu as pltpu
from jax.experimental.pallas import tpu_sc as plsc

_NC, _NS, _LANES = 2, 16, 16
_NW = _NC * _NS

_L_SC = 16384
_TL = 2048


def _make_sc_call(B, A, N, LSC):
    G = B * A
    CL = LSC // _NW
    assert LSC % _NW == 0 and CL % 128 == 0

    mesh = plsc.VectorSubcoreMesh(
        core_axis_name="c", subcore_axis_name="s",
        num_cores=_NC, num_subcores=_NS)

    @functools.partial(
        pl.kernel,
        out_type=(
            jax.ShapeDtypeStruct((B, A, LSC), jnp.float32),
            jax.ShapeDtypeStruct((B, A, LSC), jnp.int32),
        ),
        mesh=mesh,
        scratch_types=[
            pltpu.VMEM((B, CL), jnp.float32),
            pltpu.VMEM((2, N, CL), jnp.float32),
            pltpu.VMEM((B, A, CL), jnp.float32),
            pltpu.VMEM((B, A, CL), jnp.int32),
            pltpu.SemaphoreType.DMA,
            pltpu.SemaphoreType.DMA,
        ],
    )
    def sc_call(mixed_hbm, ref_hbm, omax_hbm, oidx_hbm, m_v, r_v, omax_v,
                oidx_v, sr0, sr1):
        srs = (sr0, sr1)
        wid = lax.axis_index("s") * _NC + lax.axis_index("c")
        col0 = wid * CL
        pltpu.sync_copy(mixed_hbm.at[:, pl.ds(col0, CL)], m_v)

        def ref_copy(g, k):
            return pltpu.make_async_copy(
                ref_hbm.at[pl.ds(g * N, N), pl.ds(col0, CL)],
                r_v.at[k], srs[k])

        ref_copy(0, 0).start()

        @pl.loop(0, G // 2)
        def _(t):
            for k in range(2):
                g = 2 * t + k
                gn = jnp.minimum(g + 1, G - 1)
                ref_copy(gn, 1 - k).start()
                ref_copy(g, k).wait()
                b = g // A
                a = g - b * A

                @pl.loop(0, CL // _LANES, unroll=4)
                def _(j):
                    sl = pl.ds(j * _LANES, _LANES)
                    mv = m_v[b, sl]
                    best = mv * r_v[k, 0, sl]
                    besti = jnp.zeros((_LANES,), jnp.int32)
                    for n in range(1, N):
                        p = mv * r_v[k, n, sl]
                        gt = p > best
                        besti = jnp.where(gt, jnp.int32(n), besti)
                        best = jnp.maximum(best, p)
                    omax_v[b, a, sl] = best
                    oidx_v[b, a, sl] = besti

        ref_copy(G - 1, 0).wait()
        h1 = pltpu.async_copy(omax_v, omax_hbm.at[:, :, pl.ds(col0, CL)], sr0)
        h2 = pltpu.async_copy(oidx_v, oidx_hbm.at[:, :, pl.ds(col0, CL)], sr1)
        h1.wait()
        h2.wait()

    return sc_call


def _tc_body(m_ref, r_ref, omax_ref, oidx_ref, *, N, A, B):
    b = pl.program_id(0)
    m4 = m_ref[...]
    mrow = m4[0:1]
    for bb in range(1, B):
        mrow = jnp.where(b == bb, m4[bb:bb + 1], mrow)
    for a in range(A):
        r = r_ref[0, a]
        p = mrow * r
        mx = jnp.max(p, axis=0, keepdims=True)
        row = lax.broadcasted_iota(jnp.int32, p.shape, 0)
        idx = jnp.min(jnp.where(p == mx, row, N), axis=0, keepdims=True)
        omax_ref[0, a:a + 1] = mx
        oidx_ref[0, a:a + 1] = idx


def _make_tc_call(B, A, N, L, LSC, G):
    LTC = L - LSC
    assert LTC % _TL == 0
    off = LSC // _TL
    grid = (B, LTC // _TL)

    return pl.pallas_call(
        functools.partial(_tc_body, N=N, A=A, B=B),
        grid=grid,
        in_specs=[
            pl.BlockSpec((B, _TL), lambda b, c: (0, c + off)),
            pl.BlockSpec((1, A, N, _TL), lambda b, c: (b, 0, 0, c + off)),
        ],
        out_specs=[
            pl.BlockSpec((1, A, _TL), lambda b, c: (b, 0, c + off)),
            pl.BlockSpec((1, A, _TL), lambda b, c: (b, 0, c + off)),
        ],
        out_shape=(
            jax.ShapeDtypeStruct((B, A, L), jnp.float32),
            jax.ShapeDtypeStruct((B, A, L), jnp.int32),
        ),
    )


def kernel(input_mixed, ref_panel):
    B, A, N, L = ref_panel.shape
    G = B * A
    sc_call = _make_sc_call(B, A, N, _L_SC)
    tc_call = _make_tc_call(B, A, N, L, _L_SC, G)
    ref2d = ref_panel.reshape(G * N, L)
    sc_max, sc_idx = sc_call(input_mixed, ref2d)
    tc_max, tc_idx = tc_call(input_mixed, ref_panel)
    omax = lax.dynamic_update_slice(tc_max, sc_max, (0, 0, 0))
    oidx = lax.dynamic_update_slice(tc_idx, sc_idx, (0, 0, 0))
    return omax, oidx


# speedup vs baseline: 3.0170x; 1.0087x over previous
import functools

import jax
import jax.numpy as jnp
from jax import lax
from jax.experimental import pallas as pl
from jax.experimental.pallas import tpu as pltpu
from jax.experimental.pallas import tpu_sc as plsc

_NC, _NS, _LANES = 2, 16, 16
_NW = _NC * _NS

_L_SC = 20480
_TL = 2048


def _make_sc_call(B, A, N, LSC):
    G = B * A
    CL = LSC // _NW
    assert LSC % _NW == 0 and CL % 128 == 0

    mesh = plsc.VectorSubcoreMesh(
        core_axis_name="c", subcore_axis_name="s",
        num_cores=_NC, num_subcores=_NS)

    @functools.partial(
        pl.kernel,
        out_type=(
            jax.ShapeDtypeStruct((B, A, LSC), jnp.float32),
            jax.ShapeDtypeStruct((B, A, LSC), jnp.int32),
        ),
        mesh=mesh,
        scratch_types=[
            pltpu.VMEM((B, CL), jnp.float32),
            pltpu.VMEM((2, N, CL), jnp.float32),
            pltpu.VMEM((B, A, CL), jnp.float32),
            pltpu.VMEM((B, A, CL), jnp.int32),
            pltpu.SemaphoreType.DMA,
            pltpu.SemaphoreType.DMA,
        ],
    )
    def sc_call(mixed_hbm, ref_hbm, omax_hbm, oidx_hbm, m_v, r_v, omax_v,
                oidx_v, sr0, sr1):
        srs = (sr0, sr1)
        wid = lax.axis_index("s") * _NC + lax.axis_index("c")
        col0 = wid * CL
        pltpu.sync_copy(mixed_hbm.at[:, pl.ds(col0, CL)], m_v)

        def ref_copy(g, k):
            return pltpu.make_async_copy(
                ref_hbm.at[pl.ds(g * N, N), pl.ds(col0, CL)],
                r_v.at[k], srs[k])

        ref_copy(0, 0).start()

        @pl.loop(0, G // 2)
        def _(t):
            for k in range(2):
                g = 2 * t + k
                gn = jnp.minimum(g + 1, G - 1)
                ref_copy(gn, 1 - k).start()
                ref_copy(g, k).wait()
                b = g // A
                a = g - b * A

                @pl.loop(0, CL // _LANES, unroll=4)
                def _(j):
                    sl = pl.ds(j * _LANES, _LANES)
                    mv = m_v[b, sl]
                    best = mv * r_v[k, 0, sl]
                    besti = jnp.zeros((_LANES,), jnp.int32)
                    for n in range(1, N):
                        p = mv * r_v[k, n, sl]
                        gt = p > best
                        besti = jnp.where(gt, jnp.int32(n), besti)
                        best = jnp.maximum(best, p)
                    omax_v[b, a, sl] = best
                    oidx_v[b, a, sl] = besti

        ref_copy(G - 1, 0).wait()
        h1 = pltpu.async_copy(omax_v, omax_hbm.at[:, :, pl.ds(col0, CL)], sr0)
        h2 = pltpu.async_copy(oidx_v, oidx_hbm.at[:, :, pl.ds(col0, CL)], sr1)
        h1.wait()
        h2.wait()

    return sc_call


def _tc_body(m_ref, r_ref, omax_ref, oidx_ref, *, N, A, B):
    b = pl.program_id(0)
    m4 = m_ref[...]
    mrow = m4[0:1]
    for bb in range(1, B):
        mrow = jnp.where(b == bb, m4[bb:bb + 1], mrow)
    for a in range(A):
        r = r_ref[0, a]
        p = mrow * r
        mx = jnp.max(p, axis=0, keepdims=True)
        row = lax.broadcasted_iota(jnp.int32, p.shape, 0)
        idx = jnp.min(jnp.where(p == mx, row, N), axis=0, keepdims=True)
        omax_ref[0, a:a + 1] = mx
        oidx_ref[0, a:a + 1] = idx


def _make_tc_call(B, A, N, L, LSC, G):
    LTC = L - LSC
    assert LTC % _TL == 0
    off = LSC // _TL
    grid = (B, LTC // _TL)

    return pl.pallas_call(
        functools.partial(_tc_body, N=N, A=A, B=B),
        grid=grid,
        in_specs=[
            pl.BlockSpec((B, _TL), lambda b, c: (0, c + off)),
            pl.BlockSpec((1, A, N, _TL), lambda b, c: (b, 0, 0, c + off)),
        ],
        out_specs=[
            pl.BlockSpec((1, A, _TL), lambda b, c: (b, 0, c + off)),
            pl.BlockSpec((1, A, _TL), lambda b, c: (b, 0, c + off)),
        ],
        out_shape=(
            jax.ShapeDtypeStruct((B, A, L), jnp.float32),
            jax.ShapeDtypeStruct((B, A, L), jnp.int32),
        ),
    )


def kernel(input_mixed, ref_panel):
    B, A, N, L = ref_panel.shape
    G = B * A
    sc_call = _make_sc_call(B, A, N, _L_SC)
    tc_call = _make_tc_call(B, A, N, L, _L_SC, G)
    ref2d = ref_panel.reshape(G * N, L)
    sc_max, sc_idx = sc_call(input_mixed, ref2d)
    tc_max, tc_idx = tc_call(input_mixed, ref_panel)
    omax = lax.dynamic_update_slice(tc_max, sc_max, (0, 0, 0))
    oidx = lax.dynamic_update_slice(tc_idx, sc_idx, (0, 0, 0))
    return omax, oidx


# speedup vs baseline: 3.0276x; 1.0035x over previous
import functools

import jax
import jax.numpy as jnp
from jax import lax
from jax.experimental import pallas as pl
from jax.experimental.pallas import tpu as pltpu
from jax.experimental.pallas import tpu_sc as plsc

_NC, _NS, _LANES = 2, 16, 16
_NW = _NC * _NS

_L_SC = 20480
_TL = 2048


def _make_sc_call(B, A, N, LSC):
    G = B * A
    CL = LSC // _NW
    assert LSC % _NW == 0 and CL % 128 == 0

    mesh = plsc.VectorSubcoreMesh(
        core_axis_name="c", subcore_axis_name="s",
        num_cores=_NC, num_subcores=_NS)

    @functools.partial(
        pl.kernel,
        out_type=(
            jax.ShapeDtypeStruct((B, A, LSC), jnp.float32),
            jax.ShapeDtypeStruct((B, A, LSC), jnp.int32),
        ),
        mesh=mesh,
        scratch_types=[
            pltpu.VMEM((B, CL), jnp.float32),
            pltpu.VMEM((2, N, CL), jnp.float32),
            pltpu.VMEM((B, A, CL), jnp.float32),
            pltpu.VMEM((B, A, CL), jnp.int32),
            pltpu.SemaphoreType.DMA,
            pltpu.SemaphoreType.DMA,
        ],
    )
    def sc_call(mixed_hbm, ref_hbm, omax_hbm, oidx_hbm, m_v, r_v, omax_v,
                oidx_v, sr0, sr1):
        srs = (sr0, sr1)
        wid = lax.axis_index("s") * _NC + lax.axis_index("c")
        col0 = wid * CL
        pltpu.sync_copy(mixed_hbm.at[:, pl.ds(col0, CL)], m_v)

        def ref_copy(g, k):
            return pltpu.make_async_copy(
                ref_hbm.at[pl.ds(g * N, N), pl.ds(col0, CL)],
                r_v.at[k], srs[k])

        ref_copy(0, 0).start()

        @pl.loop(0, G // 2)
        def _(t):
            for k in range(2):
                g = 2 * t + k
                gn = jnp.minimum(g + 1, G - 1)
                ref_copy(gn, 1 - k).start()
                ref_copy(g, k).wait()
                b = g // A
                a = g - b * A

                @pl.loop(0, CL // _LANES, unroll=2)
                def _(j):
                    sl = pl.ds(j * _LANES, _LANES)
                    mv = m_v[b, sl]
                    best = mv * r_v[k, 0, sl]
                    besti = jnp.zeros((_LANES,), jnp.int32)
                    for n in range(1, N):
                        p = mv * r_v[k, n, sl]
                        gt = p > best
                        besti = jnp.where(gt, jnp.int32(n), besti)
                        best = jnp.maximum(best, p)
                    omax_v[b, a, sl] = best
                    oidx_v[b, a, sl] = besti

        ref_copy(G - 1, 0).wait()
        h1 = pltpu.async_copy(omax_v, omax_hbm.at[:, :, pl.ds(col0, CL)], sr0)
        h2 = pltpu.async_copy(oidx_v, oidx_hbm.at[:, :, pl.ds(col0, CL)], sr1)
        h1.wait()
        h2.wait()

    return sc_call


def _tc_body(m_ref, r_ref, omax_ref, oidx_ref, *, N, A, B):
    b = pl.program_id(0)
    m4 = m_ref[...]
    mrow = m4[0:1]
    for bb in range(1, B):
        mrow = jnp.where(b == bb, m4[bb:bb + 1], mrow)
    for a in range(A):
        r = r_ref[0, a]
        p = mrow * r
        mx = jnp.max(p, axis=0, keepdims=True)
        row = lax.broadcasted_iota(jnp.int32, p.shape, 0)
        idx = jnp.min(jnp.where(p == mx, row, N), axis=0, keepdims=True)
        omax_ref[0, a:a + 1] = mx
        oidx_ref[0, a:a + 1] = idx


def _make_tc_call(B, A, N, L, LSC, G):
    LTC = L - LSC
    assert LTC % _TL == 0
    off = LSC // _TL
    grid = (B, LTC // _TL)

    return pl.pallas_call(
        functools.partial(_tc_body, N=N, A=A, B=B),
        grid=grid,
        in_specs=[
            pl.BlockSpec((B, _TL), lambda b, c: (0, c + off)),
            pl.BlockSpec((1, A, N, _TL), lambda b, c: (b, 0, 0, c + off)),
        ],
        out_specs=[
            pl.BlockSpec((1, A, _TL), lambda b, c: (b, 0, c + off)),
            pl.BlockSpec((1, A, _TL), lambda b, c: (b, 0, c + off)),
        ],
        out_shape=(
            jax.ShapeDtypeStruct((B, A, L), jnp.float32),
            jax.ShapeDtypeStruct((B, A, L), jnp.int32),
        ),
    )


def kernel(input_mixed, ref_panel):
    B, A, N, L = ref_panel.shape
    G = B * A
    sc_call = _make_sc_call(B, A, N, _L_SC)
    tc_call = _make_tc_call(B, A, N, L, _L_SC, G)
    ref2d = ref_panel.reshape(G * N, L)
    sc_max, sc_idx = sc_call(input_mixed, ref2d)
    tc_max, tc_idx = tc_call(input_mixed, ref_panel)
    omax = lax.dynamic_update_slice(tc_max, sc_max, (0, 0, 0))
    oidx = lax.dynamic_update_slice(tc_idx, sc_idx, (0, 0, 0))
    return omax, oidx
